# baseline (speedup 1.0000x reference)
import functools

import jax
import jax.numpy as jnp
from jax import lax
from jax.experimental import pallas as pl
from jax.experimental.pallas import tpu as pltpu
from jax.experimental.pallas import tpu_sc as plsc

B, C, T = 8, 512, 512
K = 6
P = B * T


def _tables_body(f_ref, m_ref):
    x = jnp.transpose(f_ref[0], (1, 0))
    m_ref[0, 0] = x
    s = 1
    for k in range(1, K):
        tail = jnp.broadcast_to(x[T - 1:], (s, C))
        x = jnp.maximum(x, jnp.concatenate([x[s:], tail], axis=0))
        m_ref[0, k] = x
        s *= 2


def _build_tables(feat):
    return pl.pallas_call(
        _tables_body,
        grid=(B,),
        in_specs=[pl.BlockSpec((1, C, T), lambda b: (b, 0, 0))],
        out_specs=pl.BlockSpec((1, K, T, C), lambda b: (b, 0, 0, 0)),
        out_shape=jax.ShapeDtypeStruct((B, K, T, C), jnp.float32),
    )(feat)


def _tr_body(i_ref, o_ref):
    o_ref[0] = jnp.transpose(i_ref[0], (1, 0))


def _to_bct(out_btc):
    return pl.pallas_call(
        _tr_body,
        grid=(B,),
        in_specs=[pl.BlockSpec((1, T, C), lambda b: (b, 0, 0))],
        out_specs=pl.BlockSpec((1, C, T), lambda b: (b, 0, 0)),
        out_shape=jax.ShapeDtypeStruct((B, C, T), jnp.float32),
    )(out_btc)


_NC, _NS, _L = 2, 16, 16
_NW = _NC * _NS
_PPW = P // _NW
_GRP = _PPW // _L
_NBUF = 2


def _win_off(n):
    w = jnp.full((_L,), 1, jnp.int32)
    off = jnp.full((_L,), 0, jnp.int32)
    for i, thr in enumerate((2, 4, 8, 16, 32)):
        w = jnp.where(n > thr, jnp.full((_L,), thr, jnp.int32), w)
        off = jnp.where(n > thr, jnp.full((_L,), (i + 1) * T, jnp.int32), off)
    return w, off


def _sc_body(tab_hbm, reg0_hbm, reg1_hbm, out_hbm,
             reg0_v, reg1_v, idx0, idx1, idx2, rows0, rows1, rows2,
             out0, out1, gsem0, gsem1, gsem2, osem0, osem1):
    cid = lax.axis_index("c")
    sid = lax.axis_index("s")
    wid = sid * _NC + cid
    base = wid * _PPW
    b = base // T
    t0 = base - b * T
    row_base = b * (K * T)

    pltpu.sync_copy(reg0_hbm.at[pl.ds(base, _PPW)], reg0_v)
    pltpu.sync_copy(reg1_hbm.at[pl.ds(base, _PPW)], reg1_v)

    idxb = (idx0, idx1, idx2)
    rows = (rows0, rows1, rows2)
    outs = (out0, out1)
    gsems = (gsem0, gsem1, gsem2)
    osems = (osem0, osem1)

    def fill_indices(g, idx_ref):
        tvec = t0 + g * _L + lax.iota(jnp.int32, _L)
        rl = jnp.maximum(reg0_v[pl.ds(g * _L, _L)], 0)
        rr = jnp.maximum(reg1_v[pl.ds(g * _L, _L)], 1)

        lm1 = jnp.minimum(rl, tvec)
        wl, offl = _win_off(lm1 + 1)
        rw = jnp.minimum(rr, T - tvec)
        wr, offr = _win_off(rw)

        idx_ref[pl.ds(0, _L)] = offl + row_base + (tvec - lm1)
        idx_ref[pl.ds(_L, _L)] = offl + row_base + (tvec - wl + 1)
        idx_ref[pl.ds(2 * _L, _L)] = offr + row_base + tvec
        idx_ref[pl.ds(3 * _L, _L)] = offr + row_base + (tvec + rw - wr)

    gathers = [None] * _NBUF
    stores = [None, None]
    for g in range(min(_NBUF - 1, _GRP)):
        fill_indices(g, idxb[g])
        gathers[g] = pltpu.async_copy(tab_hbm.at[idxb[g]], rows[g], gsems[g])

    for g in range(_GRP):
        s = g % _NBUF
        if g + _NBUF - 1 < _GRP:
            ns = (g + _NBUF - 1) % _NBUF
            fill_indices(g + _NBUF - 1, idxb[ns])
            gathers[ns] = pltpu.async_copy(tab_hbm.at[idxb[ns]], rows[ns], gsems[ns])
        gathers[s].wait()
        so = g % 2
        if stores[so] is not None:
            stores[so].wait()

        r = rows[s]
        o = outs[so]

        ncol = C // _L

        @plsc.parallel_loop(0, _L * ncol, unroll=8)
        def _elem(n):
            i = n // ncol
            j = n - i * ncol
            sl = pl.ds(j * _L, _L)
            left = jnp.maximum(r[i, sl], r[i + _L, sl])
            right = jnp.maximum(r[i + 2 * _L, sl], r[i + 3 * _L, sl])
            o[i, sl] = left + right

        stores[so] = pltpu.async_copy(o, out_hbm.at[pl.ds(base + g * _L, _L)], osems[so])

    for st in stores:
        if st is not None:
            st.wait()


def _sc_pool(tab_flat, reg0, reg1):
    mesh = plsc.VectorSubcoreMesh(core_axis_name="c", subcore_axis_name="s")
    fn = functools.partial(
        pl.kernel,
        mesh=mesh,
        out_type=jax.ShapeDtypeStruct((P, C), jnp.float32),
        scratch_types=[
            pltpu.VMEM((_PPW,), jnp.int32),
            pltpu.VMEM((_PPW,), jnp.int32),
            pltpu.VMEM((4 * _L,), jnp.int32),
            pltpu.VMEM((4 * _L,), jnp.int32),
            pltpu.VMEM((4 * _L,), jnp.int32),
            pltpu.VMEM((4 * _L, C), jnp.float32),
            pltpu.VMEM((4 * _L, C), jnp.float32),
            pltpu.VMEM((4 * _L, C), jnp.float32),
            pltpu.VMEM((_L, C), jnp.float32),
            pltpu.VMEM((_L, C), jnp.float32),
            pltpu.SemaphoreType.DMA,
            pltpu.SemaphoreType.DMA,
            pltpu.SemaphoreType.DMA,
            pltpu.SemaphoreType.DMA,
            pltpu.SemaphoreType.DMA,
        ],
    )(_sc_body)
    return fn(tab_flat, reg0, reg1)


@jax.jit
def kernel(feat, reg):
    tabs = _build_tables(feat)
    tab_flat = tabs.reshape(B * K * T, C)
    reg0 = reg[:, :, 0].reshape(P)
    reg1 = reg[:, :, 1].reshape(P)
    out_btc = _sc_pool(tab_flat, reg0, reg1)
    return _to_bct(out_btc.reshape(B, T, C))


# speedup vs baseline: 1.1375x; 1.1375x over previous
import functools

import jax
import jax.numpy as jnp
from jax import lax
from jax.experimental import pallas as pl
from jax.experimental.pallas import tpu as pltpu
from jax.experimental.pallas import tpu_sc as plsc

B, C, T = 8, 512, 512
K = 6
P = B * T


CH = C // 2


def _encode_keys(x):
    ub = lax.bitcast_convert_type(x.astype(jnp.bfloat16), jnp.uint16)
    ui = ub.astype(jnp.int32)
    sign = lax.shift_right_logical(ui, 15)
    return ui ^ (0x8000 | (sign * 0x7FFF))


def _pack_pair(kx):
    return kx[:, :CH] | (kx[:, CH:] << 16)


def _decode_f32(k):
    sign = lax.shift_right_logical(k, 15)
    u = k ^ jnp.where(sign == 1, 0x8000, 0xFFFF)
    ub = u.astype(jnp.uint16)
    return lax.bitcast_convert_type(ub, jnp.bfloat16).astype(jnp.float32)


def _tables_body(f_ref, m_ref):
    x = _encode_keys(jnp.transpose(f_ref[0], (1, 0)))
    m_ref[0, 0] = _pack_pair(x)
    s = 1
    for k in range(1, K):
        tail = jnp.broadcast_to(x[T - 1:], (s, C))
        x = jnp.maximum(x, jnp.concatenate([x[s:], tail], axis=0))
        m_ref[0, k] = _pack_pair(x)
        s *= 2


def _build_tables(feat):
    return pl.pallas_call(
        _tables_body,
        grid=(B,),
        in_specs=[pl.BlockSpec((1, C, T), lambda b: (b, 0, 0))],
        out_specs=pl.BlockSpec((1, K, T, CH), lambda b: (b, 0, 0, 0)),
        out_shape=jax.ShapeDtypeStruct((B, K, T, CH), jnp.int32),
    )(feat)


def _tr_body(i_ref, o_ref):
    w = i_ref[0]
    wl = w[:, :CH]
    wr = w[:, CH:]
    left = jnp.concatenate(
        [_decode_f32(wl & 0xFFFF), _decode_f32(lax.shift_right_logical(wl, 16))],
        axis=1)
    right = jnp.concatenate(
        [_decode_f32(wr & 0xFFFF), _decode_f32(lax.shift_right_logical(wr, 16))],
        axis=1)
    o_ref[0] = jnp.transpose(left + right, (1, 0))


def _to_bct(out_btc):
    return pl.pallas_call(
        _tr_body,
        grid=(B,),
        in_specs=[pl.BlockSpec((1, T, C), lambda b: (b, 0, 0))],
        out_specs=pl.BlockSpec((1, C, T), lambda b: (b, 0, 0)),
        out_shape=jax.ShapeDtypeStruct((B, C, T), jnp.float32),
    )(out_btc)


_NC, _NS, _L = 2, 16, 16
_NW = _NC * _NS
_PPW = P // _NW
_GRP = _PPW // _L
_NBUF = 2


def _win_off(n):
    w = jnp.full((_L,), 1, jnp.int32)
    off = jnp.full((_L,), 0, jnp.int32)
    for i, thr in enumerate((2, 4, 8, 16, 32)):
        w = jnp.where(n > thr, jnp.full((_L,), thr, jnp.int32), w)
        off = jnp.where(n > thr, jnp.full((_L,), (i + 1) * T, jnp.int32), off)
    return w, off


def _sc_body(tab_hbm, reg0_hbm, reg1_hbm, out_hbm,
             reg0_v, reg1_v, idx0, idx1, idx2, rows0, rows1, rows2,
             out0, out1, gsem0, gsem1, gsem2, osem0, osem1):
    cid = lax.axis_index("c")
    sid = lax.axis_index("s")
    wid = sid * _NC + cid
    base = wid * _PPW
    b = base // T
    t0 = base - b * T
    row_base = b * (K * T)

    pltpu.sync_copy(reg0_hbm.at[pl.ds(base, _PPW)], reg0_v)
    pltpu.sync_copy(reg1_hbm.at[pl.ds(base, _PPW)], reg1_v)

    idxb = (idx0, idx1, idx2)
    rows = (rows0, rows1, rows2)
    outs = (out0, out1)
    gsems = (gsem0, gsem1, gsem2)
    osems = (osem0, osem1)

    def fill_indices(g, idx_ref):
        tvec = t0 + g * _L + lax.iota(jnp.int32, _L)
        rl = jnp.maximum(reg0_v[pl.ds(g * _L, _L)], 0)
        rr = jnp.maximum(reg1_v[pl.ds(g * _L, _L)], 1)

        lm1 = jnp.minimum(rl, tvec)
        wl, offl = _win_off(lm1 + 1)
        rw = jnp.minimum(rr, T - tvec)
        wr, offr = _win_off(rw)

        idx_ref[pl.ds(0, _L)] = offl + row_base + (tvec - lm1)
        idx_ref[pl.ds(_L, _L)] = offl + row_base + (tvec - wl + 1)
        idx_ref[pl.ds(2 * _L, _L)] = offr + row_base + tvec
        idx_ref[pl.ds(3 * _L, _L)] = offr + row_base + (tvec + rw - wr)

    gathers = [None] * _NBUF
    stores = [None, None]
    for g in range(min(_NBUF - 1, _GRP)):
        fill_indices(g, idxb[g])
        gathers[g] = pltpu.async_copy(tab_hbm.at[idxb[g]], rows[g], gsems[g])

    for g in range(_GRP):
        s = g % _NBUF
        if g + _NBUF - 1 < _GRP:
            ns = (g + _NBUF - 1) % _NBUF
            fill_indices(g + _NBUF - 1, idxb[ns])
            gathers[ns] = pltpu.async_copy(tab_hbm.at[idxb[ns]], rows[ns], gsems[ns])
        gathers[s].wait()
        so = g % 2
        if stores[so] is not None:
            stores[so].wait()

        r = rows[s]
        o = outs[so]

        ncol = CH // _L
        mlo = jnp.full((_L,), 0xFFFF, jnp.int32)

        @plsc.parallel_loop(0, _L * ncol, unroll=8)
        def _elem(n):
            i = n // ncol
            j = n - i * ncol
            sl = pl.ds(j * _L, _L)
            wa = r[i, sl]
            wb = r[i + _L, sl]
            wc = r[i + 2 * _L, sl]
            wd = r[i + 3 * _L, sl]
            llo = jnp.maximum(wa & mlo, wb & mlo)
            lhi = jnp.maximum(lax.shift_right_logical(wa, 16),
                              lax.shift_right_logical(wb, 16))
            rlo = jnp.maximum(wc & mlo, wd & mlo)
            rhi = jnp.maximum(lax.shift_right_logical(wc, 16),
                              lax.shift_right_logical(wd, 16))
            o[i, sl] = llo | (lhi << 16)
            o[i, pl.ds(CH + j * _L, _L)] = rlo | (rhi << 16)

        stores[so] = pltpu.async_copy(o, out_hbm.at[pl.ds(base + g * _L, _L)], osems[so])

    for st in stores:
        if st is not None:
            st.wait()


def _sc_pool(tab_flat, reg0, reg1):
    mesh = plsc.VectorSubcoreMesh(core_axis_name="c", subcore_axis_name="s")
    fn = functools.partial(
        pl.kernel,
        mesh=mesh,
        out_type=jax.ShapeDtypeStruct((P, C), jnp.int32),
        scratch_types=[
            pltpu.VMEM((_PPW,), jnp.int32),
            pltpu.VMEM((_PPW,), jnp.int32),
            pltpu.VMEM((4 * _L,), jnp.int32),
            pltpu.VMEM((4 * _L,), jnp.int32),
            pltpu.VMEM((4 * _L,), jnp.int32),
            pltpu.VMEM((4 * _L, CH), jnp.int32),
            pltpu.VMEM((4 * _L, CH), jnp.int32),
            pltpu.VMEM((4 * _L, CH), jnp.int32),
            pltpu.VMEM((_L, C), jnp.int32),
            pltpu.VMEM((_L, C), jnp.int32),
            pltpu.SemaphoreType.DMA,
            pltpu.SemaphoreType.DMA,
            pltpu.SemaphoreType.DMA,
            pltpu.SemaphoreType.DMA,
            pltpu.SemaphoreType.DMA,
        ],
    )(_sc_body)
    return fn(tab_flat, reg0, reg1)


@jax.jit
def kernel(feat, reg):
    tabs = _build_tables(feat)
    tab_flat = tabs.reshape(B * K * T, CH)
    reg0 = reg[:, :, 0].reshape(P)
    reg1 = reg[:, :, 1].reshape(P)
    out_btc = _sc_pool(tab_flat, reg0, reg1)
    return _to_bct(out_btc.reshape(B, T, C))
